# Initial kernel scaffold; baseline (speedup 1.0000x reference)
import jax
import jax.numpy as jnp
from jax.experimental import pallas as pl


def kernel(x, edge_index, W1, W2):
    raise NotImplementedError("write your pallas kernel here")


# speedup vs baseline: 2.7144x; 2.7144x over previous
import functools

import jax
import jax.numpy as jnp
from jax import lax
from jax.experimental import pallas as pl
from jax.experimental.pallas import tpu as pltpu
from jax.experimental.pallas import tpu_sc as plsc

N = 10000
E = 320000
D = 128

NC = 2
NS = 16
NW = NC * NS

CHUNK = 128
CPT = 80
EPT = CPT * CHUNK
E_PAD = EPT * NW

ACC_ROWS = N + 112
PAD_DST = N
ZROWS = ACC_ROWS // NS
OROWS = 632
OROWS_LAST = N - 15 * OROWS

MM_BLK = 1000


def _seg_body(h_hbm, srcs_hbm, dsts_hbm, zeros_hbm, out0_hbm, out1_hbm,
              idx_s, idx_d, rows, acc, sem_g):
  c = lax.axis_index("c")
  s = lax.axis_index("s")
  wid = s * NC + c

  pltpu.sync_copy(zeros_hbm.at[pl.ds(s * ZROWS, ZROWS)],
                  acc.at[pl.ds(s * ZROWS, ZROWS)])
  pltpu.sync_copy(srcs_hbm.at[wid], idx_s)
  pltpu.sync_copy(dsts_hbm.at[wid], idx_d)
  plsc.subcore_barrier()

  def body(j, carry):
    pltpu.async_copy(h_hbm.at[idx_s.at[j]], rows, sem_g).wait()
    pltpu.sync_copy(rows, acc.at[idx_d.at[j]], add=True)
    return carry

  lax.fori_loop(0, CPT, body, 0)
  plsc.subcore_barrier()

  @pl.when(jnp.logical_and(c == 0, s < NS - 1))
  def _():
    pltpu.sync_copy(acc.at[pl.ds(s * OROWS, OROWS)],
                    out0_hbm.at[pl.ds(s * OROWS, OROWS)])

  @pl.when(jnp.logical_and(c == 0, s == NS - 1))
  def _():
    pltpu.sync_copy(acc.at[pl.ds((NS - 1) * OROWS, OROWS_LAST)],
                    out0_hbm.at[pl.ds((NS - 1) * OROWS, OROWS_LAST)])

  @pl.when(jnp.logical_and(c == 1, s < NS - 1))
  def _():
    pltpu.sync_copy(acc.at[pl.ds(s * OROWS, OROWS)],
                    out1_hbm.at[pl.ds(s * OROWS, OROWS)])

  @pl.when(jnp.logical_and(c == 1, s == NS - 1))
  def _():
    pltpu.sync_copy(acc.at[pl.ds((NS - 1) * OROWS, OROWS_LAST)],
                    out1_hbm.at[pl.ds((NS - 1) * OROWS, OROWS_LAST)])


@functools.cache
def _seg_sum_kernel():
  return pl.kernel(
      _seg_body,
      out_type=(jax.ShapeDtypeStruct((N, D), jnp.float32),
                jax.ShapeDtypeStruct((N, D), jnp.float32)),
      mesh=plsc.VectorSubcoreMesh(core_axis_name="c", subcore_axis_name="s",
                                  num_cores=NC, num_subcores=NS),
      scratch_types=[
          pltpu.VMEM((CPT, CHUNK), jnp.int32),
          pltpu.VMEM((CPT, CHUNK), jnp.int32),
          pltpu.VMEM((CHUNK, D), jnp.float32),
          pltpu.VMEM_SHARED((ACC_ROWS, D), jnp.float32),
          pltpu.SemaphoreType.DMA,
      ],
  )


def _mm_body(x_ref, w_ref, o_ref):
  o_ref[...] = jnp.dot(x_ref[...], w_ref[...],
                       preferred_element_type=jnp.float32)


def _mm(x, w):
  return pl.pallas_call(
      _mm_body,
      grid=(N // MM_BLK,),
      in_specs=[pl.BlockSpec((MM_BLK, D), lambda i: (i, 0)),
                pl.BlockSpec((D, D), lambda i: (0, 0))],
      out_specs=pl.BlockSpec((MM_BLK, D), lambda i: (i, 0)),
      out_shape=jax.ShapeDtypeStruct((N, D), jnp.float32),
  )(x, w)


def _mm_relu_add_body(p0_ref, p1_ref, w_ref, o_ref):
  h = jnp.maximum(p0_ref[...] + p1_ref[...], 0.0)
  o_ref[...] = jnp.dot(h, w_ref[...], preferred_element_type=jnp.float32)


def _mm_relu_add(p0, p1, w):
  return pl.pallas_call(
      _mm_relu_add_body,
      grid=(N // MM_BLK,),
      in_specs=[pl.BlockSpec((MM_BLK, D), lambda i: (i, 0)),
                pl.BlockSpec((MM_BLK, D), lambda i: (i, 0)),
                pl.BlockSpec((D, D), lambda i: (0, 0))],
      out_specs=pl.BlockSpec((MM_BLK, D), lambda i: (i, 0)),
      out_shape=jax.ShapeDtypeStruct((N, D), jnp.float32),
  )(p0, p1, w)


def _add_logsoftmax_body(p0_ref, p1_ref, o_ref):
  z = p0_ref[...] + p1_ref[...]
  m = jnp.max(z, axis=1, keepdims=True)
  e = jnp.exp(z - m)
  o_ref[...] = z - m - jnp.log(jnp.sum(e, axis=1, keepdims=True))


def _add_logsoftmax(p0, p1):
  return pl.pallas_call(
      _add_logsoftmax_body,
      grid=(N // MM_BLK,),
      in_specs=[pl.BlockSpec((MM_BLK, D), lambda i: (i, 0)),
                pl.BlockSpec((MM_BLK, D), lambda i: (i, 0))],
      out_specs=pl.BlockSpec((MM_BLK, D), lambda i: (i, 0)),
      out_shape=jax.ShapeDtypeStruct((N, D), jnp.float32),
  )(p0, p1)


@jax.jit
def kernel(x, edge_index, W1, W2):
  src = edge_index[0].astype(jnp.int32)
  dst = edge_index[1].astype(jnp.int32)
  srcs = jnp.pad(src, (0, E_PAD - E)).reshape(NW, CPT, CHUNK)
  dsts = jnp.pad(dst, (0, E_PAD - E),
                 constant_values=PAD_DST).reshape(NW, CPT, CHUNK)
  zeros = jnp.zeros((ACC_ROWS, D), jnp.float32)

  h1 = _mm(x, W1)
  p0, p1 = _seg_sum_kernel()(h1, srcs, dsts, zeros)
  h2 = _mm_relu_add(p0, p1, W2)
  q0, q1 = _seg_sum_kernel()(h2, srcs, dsts, zeros)
  return _add_logsoftmax(q0, q1)


# speedup vs baseline: 4.5726x; 1.6845x over previous
import functools

import jax
import jax.numpy as jnp
from jax import lax
from jax.experimental import pallas as pl
from jax.experimental.pallas import tpu as pltpu
from jax.experimental.pallas import tpu_sc as plsc

N = 10000
E = 320000
D = 128
DH = D // 2

NC = 2
NS = 16
NW = NC * NS

CHUNK = 128
CPT = 160
EPT = CPT * CHUNK
E_PAD = EPT * NS

ACC_ROWS = N + 112
PAD_DST = N
ZROWS = ACC_ROWS // NS
OROWS = 632
OROWS_LAST = N - 15 * OROWS

NBUF = 4

MM_BLK = 1000


def _seg_body(hlo_hbm, hhi_hbm, srcs_hbm, dsts_hbm, zeros_hbm,
              outlo_hbm, outhi_hbm, idx_s, idx_d, rows, sem_g, sem_s, acc):
  c = lax.axis_index("c")
  s = lax.axis_index("s")

  pltpu.sync_copy(zeros_hbm.at[pl.ds(s * ZROWS, ZROWS)],
                  acc.at[pl.ds(s * ZROWS, ZROWS)])
  pltpu.sync_copy(srcs_hbm.at[s], idx_s)
  pltpu.sync_copy(dsts_hbm.at[s], idx_d)
  plsc.subcore_barrier()

  def pump(table):
    def gather(j, k):
      pltpu.async_copy(table.at[idx_s.at[j]], rows[k], sem_g[k])

    def gather_wait(k):
      pltpu.make_async_copy(table.at[idx_s.at[0]], rows[k], sem_g[k]).wait()

    def scatter(j, k):
      pltpu.async_copy(rows[k], acc.at[idx_d.at[j]], sem_s[k], add=True)

    def scatter_wait(k):
      pltpu.make_async_copy(rows[k], acc.at[idx_d.at[0]], sem_s[k]).wait()

    gather(0, 0)
    gather(1, 1)

    def body(i, carry):
      a = i * NBUF
      for k in range(NBUF):
        j = a + k
        gather_wait(k)
        kn = (k + 2) % NBUF
        jn = j + 2

        @pl.when(jn - NBUF >= 0)
        def _():
          scatter_wait(kn)

        @pl.when(jn < CPT)
        def _():
          gather(jn, kn)

        scatter(j, k)
      return carry

    lax.fori_loop(0, CPT // NBUF, body, 0)
    scatter_wait(NBUF - 2)
    scatter_wait(NBUF - 1)

  @pl.when(c == 0)
  def _():
    pump(hlo_hbm)

  @pl.when(c == 1)
  def _():
    pump(hhi_hbm)

  plsc.subcore_barrier()

  @pl.when(jnp.logical_and(c == 0, s < NS - 1))
  def _():
    pltpu.sync_copy(acc.at[pl.ds(s * OROWS, OROWS)],
                    outlo_hbm.at[pl.ds(s * OROWS, OROWS)])

  @pl.when(jnp.logical_and(c == 0, s == NS - 1))
  def _():
    pltpu.sync_copy(acc.at[pl.ds((NS - 1) * OROWS, OROWS_LAST)],
                    outlo_hbm.at[pl.ds((NS - 1) * OROWS, OROWS_LAST)])

  @pl.when(jnp.logical_and(c == 1, s < NS - 1))
  def _():
    pltpu.sync_copy(acc.at[pl.ds(s * OROWS, OROWS)],
                    outhi_hbm.at[pl.ds(s * OROWS, OROWS)])

  @pl.when(jnp.logical_and(c == 1, s == NS - 1))
  def _():
    pltpu.sync_copy(acc.at[pl.ds((NS - 1) * OROWS, OROWS_LAST)],
                    outhi_hbm.at[pl.ds((NS - 1) * OROWS, OROWS_LAST)])


@functools.cache
def _seg_sum_kernel():
  return pl.kernel(
      _seg_body,
      out_type=(jax.ShapeDtypeStruct((N, DH), jnp.float32),
                jax.ShapeDtypeStruct((N, DH), jnp.float32)),
      mesh=plsc.VectorSubcoreMesh(core_axis_name="c", subcore_axis_name="s",
                                  num_cores=NC, num_subcores=NS),
      compiler_params=pltpu.CompilerParams(use_tc_tiling_on_sc=False),
      scratch_types=[
          pltpu.VMEM((CPT, CHUNK), jnp.int32),
          pltpu.VMEM((CPT, CHUNK), jnp.int32),
          [pltpu.VMEM((CHUNK, DH), jnp.float32)] * NBUF,
          [pltpu.SemaphoreType.DMA] * NBUF,
          [pltpu.SemaphoreType.DMA] * NBUF,
          pltpu.VMEM_SHARED((ACC_ROWS, DH), jnp.float32),
      ],
  )


def _mm_body(x_ref, w_ref, olo_ref, ohi_ref):
  r = jnp.dot(x_ref[...], w_ref[...], preferred_element_type=jnp.float32)
  olo_ref[...] = r[:, :DH]
  ohi_ref[...] = r[:, DH:]


def _mm(x, w):
  return pl.pallas_call(
      _mm_body,
      grid=(N // MM_BLK,),
      in_specs=[pl.BlockSpec((MM_BLK, D), lambda i: (i, 0)),
                pl.BlockSpec((D, D), lambda i: (0, 0))],
      out_specs=[pl.BlockSpec((MM_BLK, DH), lambda i: (i, 0)),
                 pl.BlockSpec((MM_BLK, DH), lambda i: (i, 0))],
      out_shape=[jax.ShapeDtypeStruct((N, DH), jnp.float32),
                 jax.ShapeDtypeStruct((N, DH), jnp.float32)],
  )(x, w)


def _mm_relu_body(plo_ref, phi_ref, w_ref, olo_ref, ohi_ref):
  h = jnp.maximum(jnp.concatenate([plo_ref[...], phi_ref[...]], axis=1), 0.0)
  r = jnp.dot(h, w_ref[...], preferred_element_type=jnp.float32)
  olo_ref[...] = r[:, :DH]
  ohi_ref[...] = r[:, DH:]


def _mm_relu(plo, phi, w):
  return pl.pallas_call(
      _mm_relu_body,
      grid=(N // MM_BLK,),
      in_specs=[pl.BlockSpec((MM_BLK, DH), lambda i: (i, 0)),
                pl.BlockSpec((MM_BLK, DH), lambda i: (i, 0)),
                pl.BlockSpec((D, D), lambda i: (0, 0))],
      out_specs=[pl.BlockSpec((MM_BLK, DH), lambda i: (i, 0)),
                 pl.BlockSpec((MM_BLK, DH), lambda i: (i, 0))],
      out_shape=[jax.ShapeDtypeStruct((N, DH), jnp.float32),
                 jax.ShapeDtypeStruct((N, DH), jnp.float32)],
  )(plo, phi, w)


def _logsoftmax_body(qlo_ref, qhi_ref, o_ref):
  z = jnp.concatenate([qlo_ref[...], qhi_ref[...]], axis=1)
  m = jnp.max(z, axis=1, keepdims=True)
  e = jnp.exp(z - m)
  o_ref[...] = z - m - jnp.log(jnp.sum(e, axis=1, keepdims=True))


def _logsoftmax(qlo, qhi):
  return pl.pallas_call(
      _logsoftmax_body,
      grid=(N // MM_BLK,),
      in_specs=[pl.BlockSpec((MM_BLK, DH), lambda i: (i, 0)),
                pl.BlockSpec((MM_BLK, DH), lambda i: (i, 0))],
      out_specs=pl.BlockSpec((MM_BLK, D), lambda i: (i, 0)),
      out_shape=jax.ShapeDtypeStruct((N, D), jnp.float32),
  )(qlo, qhi)


@jax.jit
def kernel(x, edge_index, W1, W2):
  src = edge_index[0].astype(jnp.int32)
  dst = edge_index[1].astype(jnp.int32)
  srcs = jnp.pad(src, (0, E_PAD - E)).reshape(NS, CPT, CHUNK)
  dsts = jnp.pad(dst, (0, E_PAD - E),
                 constant_values=PAD_DST).reshape(NS, CPT, CHUNK)
  zeros = jnp.zeros((ACC_ROWS, DH), jnp.float32)

  hlo, hhi = _mm(x, W1)
  plo, phi = _seg_sum_kernel()(hlo, hhi, srcs, dsts, zeros)
  qlo, qhi = _mm_relu(plo, phi, W2)
  rlo, rhi = _seg_sum_kernel()(qlo, qhi, srcs, dsts, zeros)
  return _logsoftmax(rlo, rhi)


# speedup vs baseline: 4.6761x; 1.0227x over previous
import functools

import jax
import jax.numpy as jnp
from jax import lax
from jax.experimental import pallas as pl
from jax.experimental.pallas import tpu as pltpu
from jax.experimental.pallas import tpu_sc as plsc

N = 10000
E = 320000
D = 128
DH = D // 2

NC = 2
NS = 16
NW = NC * NS

CHUNK = 128
CPT = 160
EPT = CPT * CHUNK
E_PAD = EPT * NS

ACC_ROWS = N + 112
PAD_DST = N
ZROWS = ACC_ROWS // NS
OROWS = 632
OROWS_LAST = N - 15 * OROWS

NBUF = 5
LOOK = 3

MM_BLK = 1000


def _seg_body(hlo_hbm, hhi_hbm, srcs_hbm, dsts_hbm, zeros_hbm,
              outlo_hbm, outhi_hbm, idx_s, idx_d, rows, sem_g, sem_s, acc):
  c = lax.axis_index("c")
  s = lax.axis_index("s")

  pltpu.sync_copy(zeros_hbm.at[pl.ds(s * ZROWS, ZROWS)],
                  acc.at[pl.ds(s * ZROWS, ZROWS)])
  pltpu.sync_copy(srcs_hbm.at[s], idx_s)
  pltpu.sync_copy(dsts_hbm.at[s], idx_d)
  plsc.subcore_barrier()

  def pump(table):
    def gather(j, k):
      pltpu.async_copy(table.at[idx_s.at[j]], rows[k], sem_g[k])

    def gather_wait(k):
      pltpu.make_async_copy(table.at[idx_s.at[0]], rows[k], sem_g[k]).wait()

    def scatter(j, k):
      pltpu.async_copy(rows[k], acc.at[idx_d.at[j]], sem_s[k], add=True)

    def scatter_wait(k):
      pltpu.make_async_copy(rows[k], acc.at[idx_d.at[0]], sem_s[k]).wait()

    def step(j, k, need_swait, issue_gather):
      gather_wait(k)
      kn = (k + LOOK) % NBUF
      if need_swait:
        scatter_wait(kn)
      if issue_gather:
        gather(j + LOOK, kn)
      scatter(j, k)

    for t in range(LOOK):
      gather(t, t)
    for k in range(NBUF):
      step(k, k, need_swait=(k + LOOK >= NBUF), issue_gather=True)

    def body(i, carry):
      a = i * NBUF
      for k in range(NBUF):
        step(a + k, k, need_swait=True, issue_gather=True)
      return carry

    lax.fori_loop(1, CPT // NBUF - 1, body, 0)

    a = CPT - NBUF
    for k in range(NBUF):
      step(a + k, k, need_swait=True, issue_gather=(k + LOOK < NBUF))
    for t in range(NBUF - LOOK):
      scatter_wait((LOOK + t) % NBUF)

  @pl.when(c == 0)
  def _():
    pump(hlo_hbm)

  @pl.when(c == 1)
  def _():
    pump(hhi_hbm)

  plsc.subcore_barrier()

  @pl.when(jnp.logical_and(c == 0, s < NS - 1))
  def _():
    pltpu.sync_copy(acc.at[pl.ds(s * OROWS, OROWS)],
                    outlo_hbm.at[pl.ds(s * OROWS, OROWS)])

  @pl.when(jnp.logical_and(c == 0, s == NS - 1))
  def _():
    pltpu.sync_copy(acc.at[pl.ds((NS - 1) * OROWS, OROWS_LAST)],
                    outlo_hbm.at[pl.ds((NS - 1) * OROWS, OROWS_LAST)])

  @pl.when(jnp.logical_and(c == 1, s < NS - 1))
  def _():
    pltpu.sync_copy(acc.at[pl.ds(s * OROWS, OROWS)],
                    outhi_hbm.at[pl.ds(s * OROWS, OROWS)])

  @pl.when(jnp.logical_and(c == 1, s == NS - 1))
  def _():
    pltpu.sync_copy(acc.at[pl.ds((NS - 1) * OROWS, OROWS_LAST)],
                    outhi_hbm.at[pl.ds((NS - 1) * OROWS, OROWS_LAST)])


@functools.cache
def _seg_sum_kernel():
  return pl.kernel(
      _seg_body,
      out_type=(jax.ShapeDtypeStruct((N, DH), jnp.float32),
                jax.ShapeDtypeStruct((N, DH), jnp.float32)),
      mesh=plsc.VectorSubcoreMesh(core_axis_name="c", subcore_axis_name="s",
                                  num_cores=NC, num_subcores=NS),
      compiler_params=pltpu.CompilerParams(use_tc_tiling_on_sc=False),
      scratch_types=[
          pltpu.VMEM((CPT, CHUNK), jnp.int32),
          pltpu.VMEM((CPT, CHUNK), jnp.int32),
          [pltpu.VMEM((CHUNK, DH), jnp.float32)] * NBUF,
          [pltpu.SemaphoreType.DMA] * NBUF,
          [pltpu.SemaphoreType.DMA] * NBUF,
          pltpu.VMEM_SHARED((ACC_ROWS, DH), jnp.float32),
      ],
  )


def _mm_body(x_ref, w_ref, olo_ref, ohi_ref):
  r = jnp.dot(x_ref[...], w_ref[...], preferred_element_type=jnp.float32)
  olo_ref[...] = r[:, :DH]
  ohi_ref[...] = r[:, DH:]


def _mm(x, w):
  return pl.pallas_call(
      _mm_body,
      grid=(N // MM_BLK,),
      in_specs=[pl.BlockSpec((MM_BLK, D), lambda i: (i, 0)),
                pl.BlockSpec((D, D), lambda i: (0, 0))],
      out_specs=[pl.BlockSpec((MM_BLK, DH), lambda i: (i, 0)),
                 pl.BlockSpec((MM_BLK, DH), lambda i: (i, 0))],
      out_shape=[jax.ShapeDtypeStruct((N, DH), jnp.float32),
                 jax.ShapeDtypeStruct((N, DH), jnp.float32)],
  )(x, w)


def _mm_relu_body(plo_ref, phi_ref, w_ref, olo_ref, ohi_ref):
  h = jnp.maximum(jnp.concatenate([plo_ref[...], phi_ref[...]], axis=1), 0.0)
  r = jnp.dot(h, w_ref[...], preferred_element_type=jnp.float32)
  olo_ref[...] = r[:, :DH]
  ohi_ref[...] = r[:, DH:]


def _mm_relu(plo, phi, w):
  return pl.pallas_call(
      _mm_relu_body,
      grid=(N // MM_BLK,),
      in_specs=[pl.BlockSpec((MM_BLK, DH), lambda i: (i, 0)),
                pl.BlockSpec((MM_BLK, DH), lambda i: (i, 0)),
                pl.BlockSpec((D, D), lambda i: (0, 0))],
      out_specs=[pl.BlockSpec((MM_BLK, DH), lambda i: (i, 0)),
                 pl.BlockSpec((MM_BLK, DH), lambda i: (i, 0))],
      out_shape=[jax.ShapeDtypeStruct((N, DH), jnp.float32),
                 jax.ShapeDtypeStruct((N, DH), jnp.float32)],
  )(plo, phi, w)


def _logsoftmax_body(qlo_ref, qhi_ref, o_ref):
  z = jnp.concatenate([qlo_ref[...], qhi_ref[...]], axis=1)
  m = jnp.max(z, axis=1, keepdims=True)
  e = jnp.exp(z - m)
  o_ref[...] = z - m - jnp.log(jnp.sum(e, axis=1, keepdims=True))


def _logsoftmax(qlo, qhi):
  return pl.pallas_call(
      _logsoftmax_body,
      grid=(N // MM_BLK,),
      in_specs=[pl.BlockSpec((MM_BLK, DH), lambda i: (i, 0)),
                pl.BlockSpec((MM_BLK, DH), lambda i: (i, 0))],
      out_specs=pl.BlockSpec((MM_BLK, D), lambda i: (i, 0)),
      out_shape=jax.ShapeDtypeStruct((N, D), jnp.float32),
  )(qlo, qhi)


@jax.jit
def kernel(x, edge_index, W1, W2):
  src = edge_index[0].astype(jnp.int32)
  dst = edge_index[1].astype(jnp.int32)
  srcs = jnp.pad(src, (0, E_PAD - E)).reshape(NS, CPT, CHUNK)
  dsts = jnp.pad(dst, (0, E_PAD - E),
                 constant_values=PAD_DST).reshape(NS, CPT, CHUNK)
  zeros = jnp.zeros((ACC_ROWS, DH), jnp.float32)

  hlo, hhi = _mm(x, W1)
  plo, phi = _seg_sum_kernel()(hlo, hhi, srcs, dsts, zeros)
  qlo, qhi = _mm_relu(plo, phi, W2)
  rlo, rhi = _seg_sum_kernel()(qlo, qhi, srcs, dsts, zeros)
  return _logsoftmax(rlo, rhi)
